# Initial kernel scaffold; baseline (speedup 1.0000x reference)
import jax
import jax.numpy as jnp
from jax.experimental import pallas as pl


def kernel(x, edge_index, W1, as1, ad1, b1, W2, as2, ad2, b2, W3, as3, ad3, b3):
    raise NotImplementedError("write your pallas kernel here")


# speedup vs baseline: 8.8670x; 8.8670x over previous
import functools

import jax
import jax.numpy as jnp
from jax import lax
from jax.experimental import pallas as pl
from jax.experimental.pallas import tpu as pltpu
from jax.experimental.pallas import tpu_sc as plsc

N = 10000
NP = 10240
E = 320000
EP = 327680
EPT = EP // 16
EB = 128
NB = EPT // EB
CHB = 16
TRASH = 10200
MBLK = 256
MG = NP // MBLK


def _store_quarters(h_ref, h):
    q = h_ref.shape[2]
    for k in range(4):
        h_ref[k] = h[:, k * q:(k + 1) * q]


def _mm1_body(x_ref, w_ref, as_ref, ad_ref, h_ref, asrc_ref, adst_ref):
    h = jnp.dot(x_ref[...], w_ref[...], preferred_element_type=jnp.float32)
    _store_quarters(h_ref, h)
    asrc_ref[...] = jnp.sum(h * as_ref[0][None, :], axis=1)[None, :]
    adst_ref[...] = jnp.sum(h * ad_ref[0][None, :], axis=1)[None, :]


def _mm1(xp, W, a_s, a_d):
    in_ch = xp.shape[1]
    h2 = W.shape[1]
    q = h2 // 4
    return pl.pallas_call(
        _mm1_body,
        grid=(MG,),
        in_specs=[
            pl.BlockSpec((MBLK, in_ch), lambda m: (m, 0)),
            pl.BlockSpec((in_ch, h2), lambda m: (0, 0)),
            pl.BlockSpec((1, h2), lambda m: (0, 0)),
            pl.BlockSpec((1, h2), lambda m: (0, 0)),
        ],
        out_specs=[
            pl.BlockSpec((4, MBLK, q), lambda m: (0, m, 0)),
            pl.BlockSpec((None, 1, MBLK), lambda m: (m, 0, 0)),
            pl.BlockSpec((None, 1, MBLK), lambda m: (m, 0, 0)),
        ],
        out_shape=[
            jax.ShapeDtypeStruct((4, NP, q), jnp.float32),
            jax.ShapeDtypeStruct((MG, 1, MBLK), jnp.float32),
            jax.ShapeDtypeStruct((MG, 1, MBLK), jnp.float32),
        ],
    )(xp, W, a_s.reshape(1, h2), a_d.reshape(1, h2))


def _norm_x(acc_ref, den_ref, b_ref):
    den = jnp.sum(den_ref[...], axis=0) + 1e-16
    xs = [jnp.maximum(acc_ref[k] / den[:, None] + b_ref[k][None, :], 0.0)
          for k in range(4)]
    return jnp.concatenate(xs, axis=1)


def _mmmid_body(acc_ref, den_ref, b_ref, w_ref, as_ref, ad_ref,
                h_ref, asrc_ref, adst_ref):
    x = _norm_x(acc_ref, den_ref, b_ref)
    h = jnp.dot(x, w_ref[...], preferred_element_type=jnp.float32)
    _store_quarters(h_ref, h)
    asrc_ref[...] = jnp.sum(h * as_ref[0][None, :], axis=1)[None, :]
    adst_ref[...] = jnp.sum(h * ad_ref[0][None, :], axis=1)[None, :]


def _mmmid(acc, denp, b_prev, W, a_s, a_d):
    out_ch = W.shape[1]
    q = out_ch // 4
    return pl.pallas_call(
        _mmmid_body,
        grid=(MG,),
        in_specs=[
            pl.BlockSpec((4, MBLK, 64), lambda m: (0, m, 0)),
            pl.BlockSpec((16, MBLK), lambda m: (0, m)),
            pl.BlockSpec((4, 64), lambda m: (0, 0)),
            pl.BlockSpec((256, out_ch), lambda m: (0, 0)),
            pl.BlockSpec((1, out_ch), lambda m: (0, 0)),
            pl.BlockSpec((1, out_ch), lambda m: (0, 0)),
        ],
        out_specs=[
            pl.BlockSpec((4, MBLK, q), lambda m: (0, m, 0)),
            pl.BlockSpec((None, 1, MBLK), lambda m: (m, 0, 0)),
            pl.BlockSpec((None, 1, MBLK), lambda m: (m, 0, 0)),
        ],
        out_shape=[
            jax.ShapeDtypeStruct((4, NP, q), jnp.float32),
            jax.ShapeDtypeStruct((MG, 1, MBLK), jnp.float32),
            jax.ShapeDtypeStruct((MG, 1, MBLK), jnp.float32),
        ],
    )(acc, denp, b_prev.reshape(4, 64), W,
      a_s.reshape(1, out_ch), a_d.reshape(1, out_ch))


def _fin_body(acc_ref, den_ref, b_ref, out_ref):
    den = jnp.sum(den_ref[...], axis=0) + 1e-16
    zs = [acc_ref[k] / den[:, None] + b_ref[k][None, :] for k in range(4)]
    z = jnp.concatenate(zs, axis=1)
    m = jnp.max(z, axis=1, keepdims=True)
    e = jnp.exp(z - m)
    out_ref[...] = e / jnp.sum(e, axis=1, keepdims=True)


def _fin(acc, denp, b):
    return pl.pallas_call(
        _fin_body,
        grid=(MG,),
        in_specs=[
            pl.BlockSpec((4, MBLK, 32), lambda m: (0, m, 0)),
            pl.BlockSpec((16, MBLK), lambda m: (0, m)),
            pl.BlockSpec((4, 32), lambda m: (0, 0)),
        ],
        out_specs=pl.BlockSpec((MBLK, 128), lambda m: (m, 0)),
        out_shape=jax.ShapeDtypeStruct((NP, 128), jnp.float32),
    )(acc, denp, b.reshape(4, 32))


@functools.lru_cache(maxsize=None)
def _make_sc(q):
    mesh = plsc.VectorSubcoreMesh(core_axis_name="c", subcore_axis_name="s",
                                  num_cores=2, num_subcores=16)

    @functools.partial(
        pl.kernel,
        out_type=[
            jax.ShapeDtypeStruct((4 * NP, q), jnp.float32),
            jax.ShapeDtypeStruct((32, NP), jnp.float32),
        ],
        mesh=mesh,
        compiler_params=pltpu.CompilerParams(needs_layout_passes=False,
                                             use_tc_tiling_on_sc=False),
        scratch_types=[
            pltpu.VMEM((NP,), jnp.float32),
            pltpu.VMEM((NP,), jnp.float32),
            pltpu.VMEM((CHB, EB), jnp.int32),
            pltpu.VMEM((CHB, EB), jnp.int32),
            pltpu.VMEM((EB,), jnp.int32),
            pltpu.VMEM((EB, q), jnp.float32),
            pltpu.VMEM((EB,), jnp.float32),
            pltpu.VMEM((NP,), jnp.float32),
            pltpu.VMEM_SHARED((NP, q), jnp.float32),
        ],
    )
    def sc_edge(hq, asrc, adst, src2f, dst2f,
                acc_out, den_out,
                asl, adl, srcc, dstc, sidx, rowb, wb, denl, acc_sh):
        c = lax.axis_index("c")
        s = lax.axis_index("s")
        rows = NP // 16

        pltpu.sync_copy(asrc, asl)
        pltpu.sync_copy(adst, adl)

        zv = jnp.zeros((16,), jnp.float32)

        def zero_rowb(i, carry):
            for k in range(q // 16):
                rowb[i, pl.ds(k * 16, 16)] = zv
            return carry

        def zero_denl(i, carry):
            denl[pl.ds(i * 16, 16)] = zv
            return carry

        lax.fori_loop(0, NP // 16, zero_denl, 0)

        for p in (0, 1):
            qidx = c + 2 * p
            lax.fori_loop(0, EB, zero_rowb, 0)
            for i in range(rows // EB):
                pltpu.sync_copy(rowb, acc_sh.at[pl.ds(s * rows + i * EB, EB)])
            plsc.subcore_barrier()

            def blk(j, carry):
                off = qidx * NP
                for k in range(EB // 16):
                    sv = srcc[j, pl.ds(k * 16, 16)]
                    dv = dstc[j, pl.ds(k * 16, 16)]
                    av = plsc.load_gather(asl, [sv])
                    bv = plsc.load_gather(adl, [dv])
                    e = av + bv
                    e = jnp.maximum(e, e * jnp.float32(0.2))
                    wv = jnp.exp(e)
                    wb[pl.ds(k * 16, 16)] = wv
                    sidx[pl.ds(k * 16, 16)] = sv + off
                    if p == 0:
                        plsc.addupdate_scatter(denl, [dv], wv)
                pltpu.sync_copy(hq.at[sidx], rowb)

                def scale(i, carry2):
                    wv16 = plsc.load_gather(
                        wb, [jnp.full((16,), i, jnp.int32)])
                    for k in range(q // 16):
                        rowb[i, pl.ds(k * 16, 16)] = (
                            rowb[i, pl.ds(k * 16, 16)] * wv16)
                    return carry2

                lax.fori_loop(0, EB, scale, 0)
                pltpu.sync_copy(rowb, acc_sh.at[dstc.at[j]], add=True)
                return carry

            def chunk(cb, carry):
                pltpu.sync_copy(src2f.at[s, pl.ds(cb * CHB, CHB)], srcc)
                pltpu.sync_copy(dst2f.at[s, pl.ds(cb * CHB, CHB)], dstc)
                lax.fori_loop(0, CHB, blk, 0)
                return carry

            lax.fori_loop(0, NB // CHB, chunk, 0)
            plsc.subcore_barrier()
            pltpu.sync_copy(
                acc_sh.at[pl.ds(s * rows, rows)],
                acc_out.at[pl.ds(qidx * NP + s * rows, rows)])

        pltpu.sync_copy(denl, den_out.at[c * 16 + s])

    return sc_edge


def _sc_edge_call(out_ch, h, a_s, a_d, src2f, dst2f):
    q = out_ch // 4
    hq = h.reshape(4 * NP, q)
    acc, denp = _make_sc(q)(hq, a_s.reshape(NP), a_d.reshape(NP),
                            src2f, dst2f)
    return acc.reshape(4, NP, q), denp


def kernel(x, edge_index, W1, as1, ad1, b1, W2, as2, ad2, b2,
           W3, as3, ad3, b3):
    src = edge_index[0].astype(jnp.int32)
    dst = edge_index[1].astype(jnp.int32)
    pad_e = EP - E
    src_p = jnp.concatenate([src, jnp.zeros((pad_e,), jnp.int32)])
    dst_p = jnp.concatenate([dst, jnp.full((pad_e,), TRASH, jnp.int32)])
    src2f = src_p.reshape(16, NB, EB)
    dst2f = dst_p.reshape(16, NB, EB)
    xp = jnp.pad(x, ((0, NP - N), (0, 0)))

    h, a_s, a_d = _mm1(xp, W1, as1, ad1)
    acc, denp = _sc_edge_call(256, h, a_s, a_d, src2f, dst2f)

    for (W, asv, adv, bprev, out_ch) in (
            (W2, as2, ad2, b1, 256),
            (W2, as2, ad2, b2, 256),
            (W3, as3, ad3, b2, 128)):
        h, a_s, a_d = _mmmid(acc, denp, bprev, W, asv, adv)
        acc, denp = _sc_edge_call(out_ch, h, a_s, a_d, src2f, dst2f)

    out = _fin(acc, denp, b3)
    return out[:N]
